# Initial kernel scaffold; baseline (speedup 1.0000x reference)
import jax
import jax.numpy as jnp
from jax.experimental import pallas as pl


def kernel(features, adj, emb_table, W1, b1, W2, b2, lw1, lb1, lw2, lb2):
    raise NotImplementedError("write your pallas kernel here")


# speedup vs baseline: 1.0066x; 1.0066x over previous
import functools

import jax
import jax.numpy as jnp
from jax import lax
from jax.experimental import pallas as pl
from jax.experimental.pallas import tpu as pltpu
from jax.experimental.pallas import tpu_sc as plsc

N = 10000
NEMB = 128


_CHUNK = 80


def _make_sc_gather(num_feat, b_pad):
    info = plsc.get_sparse_core_info()
    nw = info.num_cores * info.num_subcores
    b_per_w = b_pad // nw
    assert b_per_w % _CHUNK == 0 and b_per_w % 8 == 0
    n_chunks = b_per_w // _CHUNK
    mesh = plsc.VectorSubcoreMesh(core_axis_name="c", subcore_axis_name="s")

    @functools.partial(
        pl.kernel,
        mesh=mesh,
        out_type=jax.ShapeDtypeStruct((b_pad, NEMB), jnp.float32),
        scratch_types=[
            pltpu.VMEM((_CHUNK,), jnp.int32),
            pltpu.VMEM((_CHUNK, NEMB), jnp.float32),
            pltpu.SemaphoreType.DMA,
        ],
    )
    def gather_kernel(table_hbm, idx_hbm, out_hbm, idx_v, rows_v, sem):
        wid = lax.axis_index("s") * info.num_cores + lax.axis_index("c")
        base = wid * b_per_w
        for j in range(n_chunks):
            off = base + j * _CHUNK
            pltpu.sync_copy(idx_hbm.at[pl.ds(off, _CHUNK)], idx_v)
            pltpu.async_copy(table_hbm.at[idx_v], rows_v, sem).wait()
            pltpu.sync_copy(rows_v, out_hbm.at[pl.ds(off, _CHUNK)])

    return gather_kernel


_BM = 400


def _gcn_body(adj_ref, e_ref, w1_ref, b1_ref, w2_ref, lw1_ref, lb1_ref,
              lw2_ref, lb2_ref, b2_ref, x_ref, s_s, u_s, w_s, c_s):
    p = pl.program_id(0)
    m = pl.program_id(1)

    @pl.when(jnp.logical_and(p == 0, m == 0))
    def _init():
        s_s[...] = jnp.dot(e_ref[...], w1_ref[...],
                           preferred_element_type=jnp.float32)
        t = jnp.dot(lw1_ref[...], lw2_ref[...],
                    preferred_element_type=jnp.float32)
        w_s[...] = jnp.dot(w2_ref[...], t,
                           preferred_element_type=jnp.float32)
        c_s[...] = (jnp.dot(jnp.dot(b2_ref[...], lw1_ref[...]), lw2_ref[...])
                    + jnp.dot(lb1_ref[...], lw2_ref[...]) + lb2_ref[...])

    @pl.when(p == 0)
    def _phase0():
        h = jnp.dot(adj_ref[...], s_s[...],
                    preferred_element_type=jnp.float32) + b1_ref[...]
        r = jnp.maximum(h, 0.0)
        u = jnp.dot(r, w_s[...], preferred_element_type=jnp.float32)
        u_s[pl.ds(m * _BM, _BM), :] = u
        x_ref[...] = u

    @pl.when(p == 1)
    def _phase1():
        x_ref[...] = jnp.dot(adj_ref[...], u_s[...],
                             preferred_element_type=jnp.float32) + c_s[...]


def _gcn_pallas(adj, emb, w1, b1, w2, lw1, lb1, lw2, lb2, b2):
    n = adj.shape[0]
    num_m = n // _BM
    grid = (2, num_m)
    full = lambda shape: pl.BlockSpec(shape, lambda p, m: (0, 0))
    return pl.pallas_call(
        _gcn_body,
        grid=grid,
        in_specs=[
            pl.BlockSpec((_BM, n), lambda p, m: (m, 0)),
            full((n, NEMB)),
            full((NEMB, NEMB)),
            full((1, NEMB)),
            full((NEMB, NEMB)),
            full((NEMB, 16)),
            full((1, 16)),
            full((16, 1)),
            full((1, 1)),
            full((1, NEMB)),
        ],
        out_specs=pl.BlockSpec(
            (_BM, 1), lambda p, m: (jnp.where(p == 0, 0, m + 1), 0)),
        out_shape=jax.ShapeDtypeStruct((n + _BM, 1), jnp.float32),
        scratch_shapes=[
            pltpu.VMEM((n, NEMB), jnp.float32),
            pltpu.VMEM((n, 1), jnp.float32),
            pltpu.VMEM((NEMB, 1), jnp.float32),
            pltpu.VMEM((1, 1), jnp.float32),
        ],
        compiler_params=pltpu.CompilerParams(
            dimension_semantics=("arbitrary", "arbitrary")),
    )(adj, emb, w1, b1, w2, lw1, lb1, lw2, lb2, b2)


def _sc_gather(emb_table, idx_pad):
    return _make_sc_gather(emb_table.shape[0], idx_pad.shape[0])(
        emb_table, idx_pad)


def kernel(features, adj, emb_table, W1, b1, W2, b2, lw1, lb1, lw2, lb2):
    feats = features.astype(jnp.int32)
    b_pad = 10240
    idx_pad = jnp.concatenate(
        [feats, jnp.zeros((b_pad - N,), jnp.int32)])
    emb = _sc_gather(emb_table, idx_pad)
    user_emb = emb[:N]
    x = _gcn_pallas(adj, user_emb, W1, b1.reshape(1, -1), W2, lw1,
                    lb1.reshape(1, -1), lw2, lb2.reshape(1, 1),
                    b2.reshape(1, -1))[_BM:]
    return (x, user_emb)


# speedup vs baseline: 1.0242x; 1.0175x over previous
import functools

import jax
import jax.numpy as jnp
from jax import lax
from jax.experimental import pallas as pl
from jax.experimental.pallas import tpu as pltpu
from jax.experimental.pallas import tpu_sc as plsc

N = 10000
NEMB = 128


_CHUNK = 80


def _make_sc_gather(num_feat, b_pad):
    info = plsc.get_sparse_core_info()
    nw = info.num_cores * info.num_subcores
    b_per_w = b_pad // nw
    assert b_per_w % _CHUNK == 0 and b_per_w % 8 == 0
    n_chunks = b_per_w // _CHUNK
    mesh = plsc.VectorSubcoreMesh(core_axis_name="c", subcore_axis_name="s")

    @functools.partial(
        pl.kernel,
        mesh=mesh,
        out_type=jax.ShapeDtypeStruct((b_pad, NEMB), jnp.float32),
        scratch_types=[
            pltpu.VMEM((b_per_w,), jnp.int32),
            pltpu.VMEM((b_per_w, NEMB), jnp.float32),
        ] + [pltpu.SemaphoreType.DMA] * (2 * n_chunks),
    )
    def gather_kernel(table_hbm, idx_hbm, out_hbm, idx_v, rows_v, *sems):
        gsems, osems = sems[:n_chunks], sems[n_chunks:]
        wid = lax.axis_index("s") * info.num_cores + lax.axis_index("c")
        base = wid * b_per_w
        pltpu.sync_copy(idx_hbm.at[pl.ds(base, b_per_w)], idx_v)
        gathers = [
            pltpu.async_copy(
                table_hbm.at[idx_v.at[pl.ds(j * _CHUNK, _CHUNK)]],
                rows_v.at[pl.ds(j * _CHUNK, _CHUNK)], gsems[j])
            for j in range(n_chunks)
        ]
        writes = []
        for j in range(n_chunks):
            gathers[j].wait()
            writes.append(pltpu.async_copy(
                rows_v.at[pl.ds(j * _CHUNK, _CHUNK)],
                out_hbm.at[pl.ds(base + j * _CHUNK, _CHUNK)], osems[j]))
        for wr in writes:
            wr.wait()

    return gather_kernel


_BM = 400


def _gcn_body(adj_ref, e_ref, w1_ref, b1_ref, w2_ref, lw1_ref, lb1_ref,
              lw2_ref, lb2_ref, b2_ref, x_ref, s_s, u_s, w_s, c_s):
    p = pl.program_id(0)
    m = pl.program_id(1)

    @pl.when(jnp.logical_and(p == 0, m == 0))
    def _init():
        s_s[...] = jnp.dot(e_ref[...], w1_ref[...],
                           preferred_element_type=jnp.float32)
        t = jnp.dot(lw1_ref[...], lw2_ref[...],
                    preferred_element_type=jnp.float32)
        w_s[...] = jnp.dot(w2_ref[...], t,
                           preferred_element_type=jnp.float32)
        c_s[...] = (jnp.dot(jnp.dot(b2_ref[...], lw1_ref[...]), lw2_ref[...])
                    + jnp.dot(lb1_ref[...], lw2_ref[...]) + lb2_ref[...])

    @pl.when(p == 0)
    def _phase0():
        h = jnp.dot(adj_ref[...], s_s[...],
                    preferred_element_type=jnp.float32) + b1_ref[...]
        r = jnp.maximum(h, 0.0)
        u = jnp.dot(r, w_s[...], preferred_element_type=jnp.float32)
        u_s[pl.ds(m * _BM, _BM), :] = u
        x_ref[...] = u

    @pl.when(p == 1)
    def _phase1():
        x_ref[...] = jnp.dot(adj_ref[...], u_s[...],
                             preferred_element_type=jnp.float32) + c_s[...]


def _gcn_pallas(adj, emb, w1, b1, w2, lw1, lb1, lw2, lb2, b2):
    n = adj.shape[0]
    num_m = n // _BM
    grid = (2, num_m)
    full = lambda shape: pl.BlockSpec(shape, lambda p, m: (0, 0))
    return pl.pallas_call(
        _gcn_body,
        grid=grid,
        in_specs=[
            pl.BlockSpec((_BM, n), lambda p, m: (m, 0)),
            full((n, NEMB)),
            full((NEMB, NEMB)),
            full((1, NEMB)),
            full((NEMB, NEMB)),
            full((NEMB, 16)),
            full((1, 16)),
            full((16, 1)),
            full((1, 1)),
            full((1, NEMB)),
        ],
        out_specs=pl.BlockSpec(
            (_BM, 1), lambda p, m: (jnp.where(p == 0, 0, m + 1), 0)),
        out_shape=jax.ShapeDtypeStruct((n + _BM, 1), jnp.float32),
        scratch_shapes=[
            pltpu.VMEM((n, NEMB), jnp.float32),
            pltpu.VMEM((n, 1), jnp.float32),
            pltpu.VMEM((NEMB, 1), jnp.float32),
            pltpu.VMEM((1, 1), jnp.float32),
        ],
        compiler_params=pltpu.CompilerParams(
            dimension_semantics=("arbitrary", "arbitrary")),
    )(adj, emb, w1, b1, w2, lw1, lb1, lw2, lb2, b2)


def _sc_gather(emb_table, idx_pad):
    return _make_sc_gather(emb_table.shape[0], idx_pad.shape[0])(
        emb_table, idx_pad)


def kernel(features, adj, emb_table, W1, b1, W2, b2, lw1, lb1, lw2, lb2):
    feats = features.astype(jnp.int32)
    b_pad = 10240
    idx_pad = jnp.concatenate(
        [feats, jnp.zeros((b_pad - N,), jnp.int32)])
    emb = _sc_gather(emb_table, idx_pad)
    user_emb = emb[:N]
    x = _gcn_pallas(adj, user_emb, W1, b1.reshape(1, -1), W2, lw1,
                    lb1.reshape(1, -1), lw2, lb2.reshape(1, 1),
                    b2.reshape(1, -1))[_BM:]
    return (x, user_emb)
